# Initial kernel scaffold; baseline (speedup 1.0000x reference)
import jax
import jax.numpy as jnp
from jax.experimental import pallas as pl


def kernel(positive, negative, PT_entity_embedding, PT_relation_embedding, PT_entity_embedding2, PT_relation_embedding2):
    raise NotImplementedError("write your pallas kernel here")


# speedup vs baseline: 3.2965x; 3.2965x over previous
import functools

import jax
import jax.numpy as jnp
from jax import lax
from jax.experimental import pallas as pl
from jax.experimental.pallas import tpu as pltpu
from jax.experimental.pallas import tpu_sc as plsc

_NC = 2
_NS = 16
_NW = _NC * _NS


@functools.lru_cache(maxsize=None)
def _build_sc_gather(B, NEG, D):
    TAIL = B * (NEG + 1)
    TPW = TAIL // _NW
    C = 512
    NFULL = TPW // C
    REM = TPW - NFULL * C
    HPW = B // _NW
    assert TAIL % _NW == 0 and B % _NW == 0 and REM <= HPW

    mesh = plsc.VectorSubcoreMesh(core_axis_name="c", subcore_axis_name="s")
    f32 = jnp.float32

    @functools.partial(
        pl.kernel,
        mesh=mesh,
        out_type=[
            jax.ShapeDtypeStruct((TAIL, D), f32),
            jax.ShapeDtypeStruct((TAIL, D), f32),
            jax.ShapeDtypeStruct((B, D), f32),
            jax.ShapeDtypeStruct((B, D), f32),
            jax.ShapeDtypeStruct((B, D), f32),
            jax.ShapeDtypeStruct((B, D), f32),
        ],
        scratch_types=[
            pltpu.VMEM((TPW,), jnp.int32),
            pltpu.VMEM((C, D), f32),
            pltpu.VMEM((HPW,), jnp.int32),
            pltpu.VMEM((HPW, D), f32),
            pltpu.SemaphoreType.DMA,
        ],
    )
    def sc_gather(tail_idx, head_idx, rel_idx, e1, e2, r1, r2,
                  tail1, tail2, head1, head2, rel1, rel2,
                  idx_v, buf_v, sidx_v, sbuf_v, sem):
        wid = lax.axis_index("s") * _NC + lax.axis_index("c")
        base = wid * TPW
        pltpu.sync_copy(tail_idx.at[pl.ds(base, TPW)], idx_v)

        for tab, out in ((e1, tail1), (e2, tail2)):
            def chunk(c, carry, tab=tab, out=out):
                off = c * C
                pltpu.async_copy(tab.at[idx_v.at[pl.ds(off, C)]], buf_v, sem).wait()
                pltpu.sync_copy(buf_v, out.at[pl.ds(base + off, C)])
                return carry
            lax.fori_loop(0, NFULL, chunk, 0)
            if REM:
                off = NFULL * C
                pltpu.async_copy(tab.at[idx_v.at[pl.ds(off, REM)]], sbuf_v, sem).wait()
                pltpu.sync_copy(sbuf_v, out.at[pl.ds(base + off, REM)])

        hbase = wid * HPW
        pltpu.sync_copy(head_idx.at[pl.ds(hbase, HPW)], sidx_v)
        for tab, out in ((e1, head1), (e2, head2)):
            pltpu.async_copy(tab.at[sidx_v], sbuf_v, sem).wait()
            pltpu.sync_copy(sbuf_v, out.at[pl.ds(hbase, HPW)])
        pltpu.sync_copy(rel_idx.at[pl.ds(hbase, HPW)], sidx_v)
        for tab, out in ((r1, rel1), (r2, rel2)):
            pltpu.async_copy(tab.at[sidx_v], sbuf_v, sem).wait()
            pltpu.sync_copy(sbuf_v, out.at[pl.ds(hbase, HPW)])

    return sc_gather


def kernel(positive, negative, PT_entity_embedding, PT_relation_embedding,
           PT_entity_embedding2, PT_relation_embedding2):
    B, NEG = negative.shape
    D = PT_entity_embedding.shape[1]
    tail_idx = jnp.concatenate(
        [positive[:, 2:3], negative], axis=1).reshape(-1).astype(jnp.int32)
    head_idx = positive[:, 0].astype(jnp.int32)
    rel_idx = positive[:, 1].astype(jnp.int32)

    f = _build_sc_gather(B, NEG, D)
    tail1, tail2, head1, head2, rel1, rel2 = f(
        tail_idx, head_idx, rel_idx,
        PT_entity_embedding, PT_entity_embedding2,
        PT_relation_embedding, PT_relation_embedding2)

    PT_tail1 = tail1.reshape(B, NEG + 1, D)
    PT_tail2 = tail2.reshape(B, NEG + 1, D)
    PT_head1 = head1.reshape(B, 1, D)
    PT_head2 = head2.reshape(B, 1, D)
    PT_rel1 = rel1.reshape(B, 1, D)
    PT_rel2 = rel2.reshape(B, 1, D)
    return (PT_head2, PT_rel2, PT_tail2, PT_head1, PT_rel1, PT_tail1,
            PT_head2, PT_rel2, PT_tail2)


# speedup vs baseline: 3.3666x; 1.0213x over previous
import functools

import jax
import jax.numpy as jnp
from jax import lax
from jax.experimental import pallas as pl
from jax.experimental.pallas import tpu as pltpu
from jax.experimental.pallas import tpu_sc as plsc

_NC = 2
_NS = 16
_NW = _NC * _NS


@functools.lru_cache(maxsize=None)
def _build_sc_gather(B, NEG, D):
    TAIL = B * (NEG + 1)
    TPW = TAIL // _NW
    C = 192
    NPAIR = TPW // (2 * C)
    LEFT = TPW - NPAIR * 2 * C
    HPW = B // _NW
    assert TAIL % _NW == 0 and B % _NW == 0 and LEFT <= C and LEFT % 8 == 0

    mesh = plsc.VectorSubcoreMesh(core_axis_name="c", subcore_axis_name="s")
    f32 = jnp.float32

    @functools.partial(
        pl.kernel,
        mesh=mesh,
        out_type=[
            jax.ShapeDtypeStruct((TAIL, D), f32),
            jax.ShapeDtypeStruct((TAIL, D), f32),
            jax.ShapeDtypeStruct((B, D), f32),
            jax.ShapeDtypeStruct((B, D), f32),
            jax.ShapeDtypeStruct((B, D), f32),
            jax.ShapeDtypeStruct((B, D), f32),
        ],
        scratch_types=[
            pltpu.VMEM((TPW,), jnp.int32),
            pltpu.VMEM((C, D), f32),
            pltpu.VMEM((C, D), f32),
            pltpu.VMEM((C, D), f32),
            pltpu.VMEM((C, D), f32),
            pltpu.VMEM((HPW,), jnp.int32),
            pltpu.VMEM((HPW, D), f32),
            pltpu.SemaphoreType.DMA,
            pltpu.SemaphoreType.DMA,
            pltpu.SemaphoreType.DMA,
            pltpu.SemaphoreType.DMA,
            pltpu.SemaphoreType.DMA,
            pltpu.SemaphoreType.DMA,
            pltpu.SemaphoreType.DMA,
            pltpu.SemaphoreType.DMA,
        ],
    )
    def sc_gather(tail_idx, head_idx, rel_idx, e1, e2, r1, r2,
                  tail1, tail2, head1, head2, rel1, rel2,
                  idx_v, b1a, b1b, b2a, b2b, sidx_v, sbuf_v,
                  g1a, g1b, g2a, g2b, w1a, w1b, w2a, w2b):
        wid = lax.axis_index("s") * _NC + lax.axis_index("c")
        base = wid * TPW
        pltpu.sync_copy(tail_idx.at[pl.ds(base, TPW)], idx_v)

        streams = ((e1, tail1, (b1a, b1b), (g1a, g1b), (w1a, w1b)),
                   (e2, tail2, (b2a, b2b), (g2a, g2b), (w2a, w2b)))

        for tab, out, bufs, gs, ws in streams:
            for b in range(2):
                pltpu.async_copy(tab.at[idx_v.at[pl.ds(b * C, C)]], bufs[b], gs[b])

        def body(i, carry):
            for tab, out, bufs, gs, ws in streams:
                for b in range(2):
                    off = (2 * i + b) * C
                    pltpu.make_async_copy(
                        tab.at[idx_v.at[pl.ds(0, C)]], bufs[b], gs[b]).wait()
                    pltpu.async_copy(bufs[b], out.at[pl.ds(base + off, C)], ws[b])
            for tab, out, bufs, gs, ws in streams:
                for b in range(2):
                    @pl.when(i < NPAIR - 1)
                    def _(i=i, b=b, tab=tab, out=out, bufs=bufs, gs=gs, ws=ws):
                        pltpu.make_async_copy(
                            bufs[b], out.at[pl.ds(base, C)], ws[b]).wait()
                        pltpu.async_copy(
                            tab.at[idx_v.at[pl.ds((2 * i + b + 2) * C, C)]],
                            bufs[b], gs[b])
            return carry

        lax.fori_loop(0, NPAIR, body, 0)
        for tab, out, bufs, gs, ws in streams:
            for b in range(2):
                pltpu.make_async_copy(bufs[b], out.at[pl.ds(base, C)], ws[b]).wait()

        if LEFT:
            off = NPAIR * 2 * C
            for tab, out, bufs, gs, ws in streams:
                pltpu.async_copy(tab.at[idx_v.at[pl.ds(off, LEFT)]],
                                 bufs[0].at[pl.ds(0, LEFT)], gs[0]).wait()
                pltpu.sync_copy(bufs[0].at[pl.ds(0, LEFT)],
                                out.at[pl.ds(base + off, LEFT)])

        hbase = wid * HPW
        pltpu.sync_copy(head_idx.at[pl.ds(hbase, HPW)], sidx_v)
        for tab, out in ((e1, head1), (e2, head2)):
            pltpu.async_copy(tab.at[sidx_v], sbuf_v, g1a).wait()
            pltpu.sync_copy(sbuf_v, out.at[pl.ds(hbase, HPW)])
        pltpu.sync_copy(rel_idx.at[pl.ds(hbase, HPW)], sidx_v)
        for tab, out in ((r1, rel1), (r2, rel2)):
            pltpu.async_copy(tab.at[sidx_v], sbuf_v, g1a).wait()
            pltpu.sync_copy(sbuf_v, out.at[pl.ds(hbase, HPW)])

    return sc_gather


def kernel(positive, negative, PT_entity_embedding, PT_relation_embedding,
           PT_entity_embedding2, PT_relation_embedding2):
    B, NEG = negative.shape
    D = PT_entity_embedding.shape[1]
    tail_idx = jnp.concatenate(
        [positive[:, 2:3], negative], axis=1).reshape(-1).astype(jnp.int32)
    head_idx = positive[:, 0].astype(jnp.int32)
    rel_idx = positive[:, 1].astype(jnp.int32)

    f = _build_sc_gather(B, NEG, D)
    tail1, tail2, head1, head2, rel1, rel2 = f(
        tail_idx, head_idx, rel_idx,
        PT_entity_embedding, PT_entity_embedding2,
        PT_relation_embedding, PT_relation_embedding2)

    PT_tail1 = tail1.reshape(B, NEG + 1, D)
    PT_tail2 = tail2.reshape(B, NEG + 1, D)
    PT_head1 = head1.reshape(B, 1, D)
    PT_head2 = head2.reshape(B, 1, D)
    PT_rel1 = rel1.reshape(B, 1, D)
    PT_rel2 = rel2.reshape(B, 1, D)
    return (PT_head2, PT_rel2, PT_tail2, PT_head1, PT_rel1, PT_tail1,
            PT_head2, PT_rel2, PT_tail2)


# speedup vs baseline: 5.0900x; 1.5119x over previous
import functools

import jax
import jax.numpy as jnp
from jax import lax
from jax.experimental import pallas as pl
from jax.experimental.pallas import tpu as pltpu
from jax.experimental.pallas import tpu_sc as plsc

_NC = 2
_NS = 16
_NW = _NC * _NS


@functools.lru_cache(maxsize=None)
def _build_sc_gather(B, NEG, D):
    T = NEG + 1
    TP = (T + 7) // 8 * 8
    BPW = B // _NW
    IPW = BPW * TP
    HPW = B // _NW
    assert B % _NW == 0

    mesh = plsc.VectorSubcoreMesh(core_axis_name="c", subcore_axis_name="s")
    f32 = jnp.float32

    @functools.partial(
        pl.kernel,
        mesh=mesh,
        out_type=[
            jax.ShapeDtypeStruct((B, T, D), f32),
            jax.ShapeDtypeStruct((B, T, D), f32),
            jax.ShapeDtypeStruct((B, D), f32),
            jax.ShapeDtypeStruct((B, D), f32),
            jax.ShapeDtypeStruct((B, D), f32),
            jax.ShapeDtypeStruct((B, D), f32),
        ],
        scratch_types=[
            pltpu.VMEM((IPW,), jnp.int32),
            pltpu.VMEM((T, D), f32),
            pltpu.VMEM((T, D), f32),
            pltpu.VMEM((HPW,), jnp.int32),
            pltpu.VMEM((HPW, D), f32),
            pltpu.SemaphoreType.DMA,
            pltpu.SemaphoreType.DMA,
            pltpu.SemaphoreType.DMA,
            pltpu.SemaphoreType.DMA,
        ],
    )
    def sc_gather(tail_idx, head_idx, rel_idx, e1, e2, r1, r2,
                  tail1, tail2, head1, head2, rel1, rel2,
                  idx_v, bufa, bufb, sidx_v, sbuf_v,
                  ga, gb, wa, wb):
        wid = lax.axis_index("s") * _NC + lax.axis_index("c")
        b0 = wid * BPW
        pltpu.sync_copy(tail_idx.at[pl.ds(wid * IPW, IPW)], idx_v)

        pltpu.async_copy(e1.at[idx_v.at[pl.ds(0, T)]], bufa, ga)
        pltpu.async_copy(e2.at[idx_v.at[pl.ds(0, T)]], bufb, gb)

        def body(i, carry):
            pltpu.make_async_copy(e1.at[idx_v.at[pl.ds(0, T)]], bufa, ga).wait()
            pltpu.async_copy(bufa, tail1.at[b0 + i], wa)
            pltpu.make_async_copy(e2.at[idx_v.at[pl.ds(0, T)]], bufb, gb).wait()
            pltpu.async_copy(bufb, tail2.at[b0 + i], wb)

            @pl.when(i < BPW - 1)
            def _():
                off = (i + 1) * TP
                pltpu.make_async_copy(bufa, tail1.at[b0], wa).wait()
                pltpu.async_copy(e1.at[idx_v.at[pl.ds(off, T)]], bufa, ga)
                pltpu.make_async_copy(bufb, tail2.at[b0], wb).wait()
                pltpu.async_copy(e2.at[idx_v.at[pl.ds(off, T)]], bufb, gb)
            return carry

        lax.fori_loop(0, BPW, body, 0)
        pltpu.make_async_copy(bufa, tail1.at[b0], wa).wait()
        pltpu.make_async_copy(bufb, tail2.at[b0], wb).wait()

        hbase = wid * HPW
        pltpu.sync_copy(head_idx.at[pl.ds(hbase, HPW)], sidx_v)
        for tab, out in ((e1, head1), (e2, head2)):
            pltpu.async_copy(tab.at[sidx_v], sbuf_v, ga).wait()
            pltpu.sync_copy(sbuf_v, out.at[pl.ds(hbase, HPW)])
        pltpu.sync_copy(rel_idx.at[pl.ds(hbase, HPW)], sidx_v)
        for tab, out in ((r1, rel1), (r2, rel2)):
            pltpu.async_copy(tab.at[sidx_v], sbuf_v, ga).wait()
            pltpu.sync_copy(sbuf_v, out.at[pl.ds(hbase, HPW)])

    return sc_gather


def kernel(positive, negative, PT_entity_embedding, PT_relation_embedding,
           PT_entity_embedding2, PT_relation_embedding2):
    B, NEG = negative.shape
    D = PT_entity_embedding.shape[1]
    T = NEG + 1
    TP = (T + 7) // 8 * 8
    tail_idx = jnp.concatenate(
        [positive[:, 2:3], negative,
         jnp.zeros((B, TP - T), dtype=positive.dtype)],
        axis=1).reshape(-1).astype(jnp.int32)
    head_idx = positive[:, 0].astype(jnp.int32)
    rel_idx = positive[:, 1].astype(jnp.int32)

    f = _build_sc_gather(B, NEG, D)
    PT_tail1, PT_tail2, head1, head2, rel1, rel2 = f(
        tail_idx, head_idx, rel_idx,
        PT_entity_embedding, PT_entity_embedding2,
        PT_relation_embedding, PT_relation_embedding2)

    PT_head1 = head1.reshape(B, 1, D)
    PT_head2 = head2.reshape(B, 1, D)
    PT_rel1 = rel1.reshape(B, 1, D)
    PT_rel2 = rel2.reshape(B, 1, D)
    return (PT_head2, PT_rel2, PT_tail2, PT_head1, PT_rel1, PT_tail1,
            PT_head2, PT_rel2, PT_tail2)


# speedup vs baseline: 8.6279x; 1.6951x over previous
import functools

import jax
import jax.numpy as jnp
from jax import lax
from jax.experimental import pallas as pl
from jax.experimental.pallas import tpu as pltpu
from jax.experimental.pallas import tpu_sc as plsc

_NC = 2
_NS = 16
_NW = _NC * _NS


@functools.lru_cache(maxsize=None)
def _build_sc_gather(B, NEG, D):
    T = NEG + 1
    TAIL = T * B
    TPW = (T - 1) * B // _NW
    C = 256
    NPAIR = TPW // (2 * C)
    XPW = B // _NW
    HPW = B // _NW
    assert (T - 1) * B % _NW == 0 and TPW % (2 * C) == 0 and B % _NW == 0

    mesh = plsc.VectorSubcoreMesh(core_axis_name="c", subcore_axis_name="s")
    f32 = jnp.float32

    @functools.partial(
        pl.kernel,
        mesh=mesh,
        out_type=[
            jax.ShapeDtypeStruct((TAIL, D), f32),
            jax.ShapeDtypeStruct((TAIL, D), f32),
            jax.ShapeDtypeStruct((TAIL, D), f32),
            jax.ShapeDtypeStruct((B, D), f32),
            jax.ShapeDtypeStruct((B, D), f32),
            jax.ShapeDtypeStruct((B, D), f32),
            jax.ShapeDtypeStruct((B, D), f32),
            jax.ShapeDtypeStruct((B, D), f32),
            jax.ShapeDtypeStruct((B, D), f32),
        ],
        scratch_types=[
            pltpu.VMEM((TPW,), jnp.int32),
            pltpu.VMEM((C, D), f32),
            pltpu.VMEM((C, D), f32),
            pltpu.VMEM((HPW,), jnp.int32),
            pltpu.VMEM((HPW, D), f32),
            pltpu.SemaphoreType.DMA,
            pltpu.SemaphoreType.DMA,
            pltpu.SemaphoreType.DMA,
            pltpu.SemaphoreType.DMA,
        ],
    )
    def sc_gather(tail_idx, head_idx, rel_idx, e1, e2, r1, r2,
                  tail1, tail2a, tail2b, head1, head2a, head2b,
                  rel1, rel2a, rel2b,
                  idx_v, buf0, buf1, sidx_v, sbuf_v,
                  g0, g1, w0, w1):
        wid = lax.axis_index("s") * _NC + lax.axis_index("c")
        base = wid * TPW
        bufs = (buf0, buf1)
        gs = (g0, g1)
        ws = (w0, w1)
        pltpu.sync_copy(tail_idx.at[pl.ds(base, TPW)], idx_v)

        def run_table(tab, outs):
            for b in range(2):
                pltpu.async_copy(tab.at[idx_v.at[pl.ds(b * C, C)]], bufs[b], gs[b])

            def body(i, carry):
                for b in range(2):
                    off = (2 * i + b) * C
                    pltpu.make_async_copy(
                        tab.at[idx_v.at[pl.ds(0, C)]], bufs[b], gs[b]).wait()
                    for o in outs:
                        pltpu.async_copy(bufs[b], o.at[pl.ds(base + off, C)], ws[b])
                for b in range(2):
                    @pl.when(i < NPAIR - 1)
                    def _(i=i, b=b):
                        for o in outs:
                            pltpu.make_async_copy(
                                bufs[b], outs[0].at[pl.ds(base, C)], ws[b]).wait()
                        pltpu.async_copy(
                            tab.at[idx_v.at[pl.ds((2 * i + b + 2) * C, C)]],
                            bufs[b], gs[b])
                return carry

            lax.fori_loop(0, NPAIR, body, 0)
            for b in range(2):
                for o in outs:
                    pltpu.make_async_copy(
                        bufs[b], outs[0].at[pl.ds(base, C)], ws[b]).wait()

        run_table(e1, (tail1,))
        run_table(e2, (tail2a, tail2b))

        xoff = (T - 1) * B + wid * XPW
        pltpu.sync_copy(tail_idx.at[pl.ds(xoff, XPW)], sidx_v)
        for tab, outs in ((e1, (tail1,)), (e2, (tail2a, tail2b))):
            pltpu.async_copy(tab.at[sidx_v], sbuf_v, g0).wait()
            for o in outs:
                pltpu.sync_copy(sbuf_v, o.at[pl.ds(xoff, XPW)])

        hbase = wid * HPW
        pltpu.sync_copy(head_idx.at[pl.ds(hbase, HPW)], sidx_v)
        for tab, outs in ((e1, (head1,)), (e2, (head2a, head2b))):
            pltpu.async_copy(tab.at[sidx_v], sbuf_v, g0).wait()
            for o in outs:
                pltpu.sync_copy(sbuf_v, o.at[pl.ds(hbase, HPW)])
        pltpu.sync_copy(rel_idx.at[pl.ds(hbase, HPW)], sidx_v)
        for tab, outs in ((r1, (rel1,)), (r2, (rel2a, rel2b))):
            pltpu.async_copy(tab.at[sidx_v], sbuf_v, g0).wait()
            for o in outs:
                pltpu.sync_copy(sbuf_v, o.at[pl.ds(hbase, HPW)])

    return sc_gather


def kernel(positive, negative, PT_entity_embedding, PT_relation_embedding,
           PT_entity_embedding2, PT_relation_embedding2):
    B, NEG = negative.shape
    D = PT_entity_embedding.shape[1]
    T = NEG + 1
    tail_idx = jnp.concatenate(
        [positive[:, 2:3], negative], axis=1).T.reshape(-1).astype(jnp.int32)
    head_idx = positive[:, 0].astype(jnp.int32)
    rel_idx = positive[:, 1].astype(jnp.int32)

    f = _build_sc_gather(B, NEG, D)
    (tail1, tail2a, tail2b, head1, head2a, head2b,
     rel1, rel2a, rel2b) = f(
        tail_idx, head_idx, rel_idx,
        PT_entity_embedding, PT_entity_embedding2,
        PT_relation_embedding, PT_relation_embedding2)

    def as_tail(x):
        return x.reshape(T, B, D).transpose(1, 0, 2)

    def as_one(x):
        return x.reshape(B, 1, D)

    return (as_one(head2a), as_one(rel2a), as_tail(tail2a),
            as_one(head1), as_one(rel1), as_tail(tail1),
            as_one(head2b), as_one(rel2b), as_tail(tail2b))
